# baseline (speedup 1.0000x reference)
import jax
import jax.numpy as jnp
from jax import lax
from jax.experimental import pallas as pl
from jax.experimental.pallas import tpu as pltpu
from jax.experimental.pallas import tpu_sc as plsc

NC = 2
NS = 16
SLAB = 128
NBUF = 5
KLOOK = 2
DEGW = 8


def _proj_body(x_ref, wl_ref, wr_ref, b_ref, ys_ref, self_ref):
    xx = x_ref[...]
    y = jnp.dot(xx, wl_ref[...], preferred_element_type=jnp.float32)
    half = y.shape[1] // 2
    ys_ref[0] = y[:, :half]
    ys_ref[1] = y[:, half:]
    self_ref[...] = (
        jnp.dot(xx, wr_ref[...], preferred_element_type=jnp.float32) + b_ref[...]
    )


def _mid_body(s1_ref, deg_ref, self1_ref, wl_ref, wr_ref, b_ref, y2_ref,
              self2_ref):
    nn = self1_ref.shape[0]
    d = deg_ref[0, :nn, 0:1] + deg_ref[1, :nn, 0:1]
    rec = 1.0 / jnp.maximum(d, 1.0)
    s1 = jnp.concatenate([s1_ref[0, :nn], s1_ref[1, :nn]], axis=1)
    h = jnp.maximum(s1 * rec + self1_ref[...], 0.0)
    y2_ref[...] = jnp.dot(h, wl_ref[...], preferred_element_type=jnp.float32)
    self2_ref[...] = (
        jnp.dot(h, wr_ref[...], preferred_element_type=jnp.float32) + b_ref[...]
    )


def _final_body(s2_ref, deg_ref, self2_ref, out_ref):
    nn = self2_ref.shape[0]
    d = deg_ref[0, :nn, 0:1] + deg_ref[1, :nn, 0:1]
    rec = 1.0 / jnp.maximum(d, 1.0)
    z = (s2_ref[0, :nn] + s2_ref[1, :nn]) * rec + self2_ref[...]
    m = jnp.max(z, axis=1, keepdims=True)
    zs = z - m
    lse = jnp.log(jnp.sum(jnp.exp(zs), axis=1, keepdims=True))
    out_ref[...] = zs - lse


def _npad(n_nodes):
    return ((n_nodes + 1 + 127) // 128) * 128


def _segsum_loop(y_ref, srcbuf, dstbuf, rows, g_sems, s_sems,
                 agg_sh, n_slabs, deg=None):
    G = n_slabs

    def gidx(g):
        return srcbuf.at[g]

    def sidx(g):
        return dstbuf.at[g]

    for b in range(NBUF):
        pltpu.async_copy(y_ref.at[gidx(b)], rows[b], g_sems[b])

    def group(i, carry):
        for b in range(NBUF):
            g = i * NBUF + b
            pltpu.make_async_copy(y_ref.at[gidx(g)], rows[b],
                                  g_sems[b]).wait()
            pltpu.async_copy(rows[b], agg_sh.at[sidx(g)], s_sems[b],
                             add=True)
            if deg is not None:
                ones_v, deg_sh, d_sem, cond_fn = deg

                @pl.when(cond_fn(g))
                def _():
                    pltpu.async_copy(ones_v, deg_sh.at[sidx(g)], d_sem,
                                     add=True)

            b2 = (b + KLOOK) % NBUF

            @pl.when((g >= NBUF - KLOOK) & (g + KLOOK < G))
            def _():
                pltpu.make_async_copy(rows[b2], agg_sh.at[sidx(0)],
                                      s_sems[b2]).wait()
                pltpu.async_copy(y_ref.at[gidx(g + KLOOK)], rows[b2],
                                 g_sems[b2])
        return carry

    lax.fori_loop(0, G // NBUF, group, 0)
    for b in range(NBUF):
        pltpu.make_async_copy(rows[b], agg_sh.at[sidx(0)], s_sems[b]).wait()
    if deg is not None:
        ones_v, deg_sh, d_sem, _ = deg

        def dwait(i, carry):
            pltpu.make_async_copy(ones_v, deg_sh.at[sidx(0)], d_sem).wait()
            return carry

        lax.fori_loop(0, G // 2, dwait, 0)


def _make_sc_layer1(n_nodes, half, slabs_per_tile):
    G = slabs_per_tile
    NPAD = _npad(n_nodes)
    ZR = NPAD // NS

    def body(*args):
        (ys_hbm, src_hbm, dst_hbm, zd_hbm, z16_hbm, ones_hbm,
         out_hbm, deg_hbm, srcbuf, dstbuf) = args[:10]
        rows = list(args[10:10 + NBUF])
        ones_v, agg_sh, deg_sh = args[10 + NBUF:13 + NBUF]
        g_sems = list(args[13 + NBUF:13 + 2 * NBUF])
        s_sems = list(args[13 + 2 * NBUF:13 + 3 * NBUF])
        dsem = args[13 + 3 * NBUF]
        c = lax.axis_index("c")
        s = lax.axis_index("s")

        pltpu.sync_copy(zd_hbm, agg_sh.at[pl.ds(s * ZR, ZR)])
        pltpu.sync_copy(z16_hbm, deg_sh.at[pl.ds(s * ZR, ZR)])
        pltpu.sync_copy(ones_hbm, ones_v)
        pltpu.sync_copy(src_hbm.at[pl.ds(s * G, G)], srcbuf)
        pltpu.sync_copy(dst_hbm.at[pl.ds(s * G, G)], dstbuf)
        plsc.subcore_barrier()

        hG = G // 2

        def cond_fn(g):
            return lax.select(c == 0, g < hG, g >= hG)

        _segsum_loop(ys_hbm.at[c], srcbuf, dstbuf, rows, g_sems, s_sems,
                     agg_sh, G, deg=(ones_v, deg_sh, dsem, cond_fn))

        plsc.subcore_barrier()
        pltpu.sync_copy(agg_sh.at[pl.ds(s * ZR, ZR)],
                        out_hbm.at[c, pl.ds(s * ZR, ZR)])
        pltpu.sync_copy(deg_sh.at[pl.ds(s * ZR, ZR)],
                        deg_hbm.at[c, pl.ds(s * ZR, ZR)])

    out_type = [
        jax.ShapeDtypeStruct((NC, NPAD, half), jnp.float32),
        jax.ShapeDtypeStruct((NC, NPAD, DEGW), jnp.float32),
    ]
    scratch = (
        [pltpu.VMEM((G, SLAB), jnp.int32)] * 2
        + [pltpu.VMEM((SLAB, half), jnp.float32)] * NBUF
        + [pltpu.VMEM((SLAB, DEGW), jnp.float32)]
        + [pltpu.VMEM_SHARED((NPAD, half), jnp.float32)]
        + [pltpu.VMEM_SHARED((NPAD, DEGW), jnp.float32)]
        + [pltpu.SemaphoreType.DMA] * (2 * NBUF + 1)
    )
    mesh = plsc.VectorSubcoreMesh(core_axis_name="c", subcore_axis_name="s")
    return pl.kernel(
        body, out_type=out_type, mesh=mesh, scratch_types=scratch,
        compiler_params=pltpu.CompilerParams(use_tc_tiling_on_sc=False))


def _make_sc_layer2(n_nodes, d, slabs_per_tile):
    G = slabs_per_tile
    NPAD = _npad(n_nodes)
    ZR = NPAD // NS

    def body(*args):
        y_hbm, src_hbm, dst_hbm, zd_hbm, out_hbm, srcbuf, dstbuf = args[:7]
        rows = list(args[7:7 + NBUF])
        agg_sh = args[7 + NBUF]
        g_sems = list(args[8 + NBUF:8 + 2 * NBUF])
        s_sems = list(args[8 + 2 * NBUF:8 + 3 * NBUF])
        c = lax.axis_index("c")
        s = lax.axis_index("s")
        tid = c * NS + s

        pltpu.sync_copy(zd_hbm, agg_sh.at[pl.ds(s * ZR, ZR)])
        pltpu.sync_copy(src_hbm.at[pl.ds(tid * G, G)], srcbuf)
        pltpu.sync_copy(dst_hbm.at[pl.ds(tid * G, G)], dstbuf)
        plsc.subcore_barrier()

        _segsum_loop(y_hbm, srcbuf, dstbuf, rows, g_sems, s_sems, agg_sh, G)

        plsc.subcore_barrier()
        pltpu.sync_copy(agg_sh.at[pl.ds(s * ZR, ZR)],
                        out_hbm.at[c, pl.ds(s * ZR, ZR)])

    out_type = jax.ShapeDtypeStruct((NC, NPAD, d), jnp.float32)
    scratch = (
        [pltpu.VMEM((G, SLAB), jnp.int32)] * 2
        + [pltpu.VMEM((SLAB, d), jnp.float32)] * NBUF
        + [pltpu.VMEM_SHARED((NPAD, d), jnp.float32)]
        + [pltpu.SemaphoreType.DMA] * (2 * NBUF)
    )
    mesh = plsc.VectorSubcoreMesh(core_axis_name="c", subcore_axis_name="s")
    return pl.kernel(
        body, out_type=out_type, mesh=mesh, scratch_types=scratch,
        compiler_params=pltpu.CompilerParams(use_tc_tiling_on_sc=False))


def kernel(x, edge_index, W1l, W1r, b1, W2l, W2r, b2):
    n, d_in = x.shape
    d_hid = W1l.shape[1]
    n_cls = W2l.shape[1]
    e = edge_index.shape[1]

    src = edge_index[0].astype(jnp.int32)
    dst = edge_index[1].astype(jnp.int32)

    unit = NC * NS * SLAB * NBUF
    e_pad = -(-e // unit) * unit
    pad = e_pad - e
    npad = _npad(n)
    pad_ar = jnp.arange(pad, dtype=jnp.int32)
    src_p = jnp.concatenate([src, pad_ar % n]).reshape(-1, SLAB)
    dst_p = jnp.concatenate([dst, n + pad_ar % (npad - n)]).reshape(-1, SLAB)
    g1 = e_pad // (NS * SLAB)
    g2 = e_pad // (NC * NS * SLAB)

    zr = npad // NS
    half = d_hid // 2
    z_half = jnp.zeros((zr, half), jnp.float32)
    z_cls = jnp.zeros((zr, n_cls), jnp.float32)
    z16 = jnp.zeros((zr, DEGW), jnp.float32)
    ones16 = jnp.ones((SLAB, DEGW), jnp.float32)

    y1s, self1 = pl.pallas_call(
        _proj_body,
        out_shape=[
            jax.ShapeDtypeStruct((NC, n, half), jnp.float32),
            jax.ShapeDtypeStruct((n, d_hid), jnp.float32),
        ],
    )(x, W1l, W1r, b1.reshape(1, -1))

    sc1 = _make_sc_layer1(n, half, g1)
    s1p, degp = sc1(y1s, src_p, dst_p, z_half, z16, ones16)

    y2, self2 = pl.pallas_call(
        _mid_body,
        out_shape=[
            jax.ShapeDtypeStruct((n, n_cls), jnp.float32),
            jax.ShapeDtypeStruct((n, n_cls), jnp.float32),
        ],
    )(s1p, degp, self1, W2l, W2r, b2.reshape(1, -1))

    sc2 = _make_sc_layer2(n, n_cls, g2)
    s2p = sc2(y2, src_p, dst_p, z_cls)

    out = pl.pallas_call(
        _final_body,
        out_shape=jax.ShapeDtypeStruct((n, n_cls), jnp.float32),
    )(s2p, degp, self2)
    return out


# speedup vs baseline: 1.2257x; 1.2257x over previous
import jax
import jax.numpy as jnp
from jax import lax
from jax.experimental import pallas as pl
from jax.experimental.pallas import tpu as pltpu
from jax.experimental.pallas import tpu_sc as plsc

NC = 2
NS = 16
SLAB = 128
NBUF = 4
KLOOK = 2
DEGW = 8


def _proj_body(x_ref, wl_ref, wr_ref, b_ref, ys_ref, self_ref):
    xx = x_ref[...]
    y = jnp.dot(xx, wl_ref[...], preferred_element_type=jnp.float32)
    half = y.shape[1] // 2
    yb = y.astype(jnp.bfloat16)
    ys_ref[0] = yb[:, :half]
    ys_ref[1] = yb[:, half:]
    self_ref[...] = (
        jnp.dot(xx, wr_ref[...], preferred_element_type=jnp.float32) + b_ref[...]
    )


def _mid_body(s1_ref, deg_ref, self1_ref, wl_ref, wr_ref, b_ref, y2_ref,
              self2_ref):
    nn = self1_ref.shape[0]
    d = deg_ref[0, :nn, 0:1] + deg_ref[1, :nn, 0:1]
    rec = 1.0 / jnp.maximum(d, 1.0)
    s1 = jnp.concatenate([s1_ref[0, :nn], s1_ref[1, :nn]],
                         axis=1).astype(jnp.float32)
    h = jnp.maximum(s1 * rec + self1_ref[...], 0.0)
    y2_ref[...] = jnp.dot(h, wl_ref[...],
                          preferred_element_type=jnp.float32).astype(jnp.bfloat16)
    self2_ref[...] = (
        jnp.dot(h, wr_ref[...], preferred_element_type=jnp.float32) + b_ref[...]
    )


def _final_body(s2_ref, deg_ref, self2_ref, out_ref):
    nn = self2_ref.shape[0]
    d = deg_ref[0, :nn, 0:1] + deg_ref[1, :nn, 0:1]
    rec = 1.0 / jnp.maximum(d, 1.0)
    s2 = s2_ref[0, :nn].astype(jnp.float32) + s2_ref[1, :nn].astype(jnp.float32)
    z = s2 * rec + self2_ref[...]
    m = jnp.max(z, axis=1, keepdims=True)
    zs = z - m
    lse = jnp.log(jnp.sum(jnp.exp(zs), axis=1, keepdims=True))
    out_ref[...] = zs - lse


def _npad(n_nodes):
    return ((n_nodes + 1 + 127) // 128) * 128


def _segsum_loop(y_ref, srcbuf, dstbuf, rows, g_sems, s_sems,
                 agg_sh, n_slabs, deg=None):
    G = n_slabs

    def gidx(g):
        return srcbuf.at[g]

    def sidx(g):
        return dstbuf.at[g]

    for b in range(NBUF):
        pltpu.async_copy(y_ref.at[gidx(b)], rows[b], g_sems[b])

    def group(i, carry):
        for b in range(NBUF):
            g = i * NBUF + b
            pltpu.make_async_copy(y_ref.at[gidx(g)], rows[b],
                                  g_sems[b]).wait()
            pltpu.async_copy(rows[b], agg_sh.at[sidx(g)], s_sems[b],
                             add=True)
            if deg is not None:
                ones_v, deg_sh, d_sem, cond_fn = deg

                @pl.when(cond_fn(g))
                def _():
                    pltpu.async_copy(ones_v, deg_sh.at[sidx(g)], d_sem,
                                     add=True)

            b2 = (b + KLOOK) % NBUF

            @pl.when((g >= NBUF - KLOOK) & (g + KLOOK < G))
            def _():
                pltpu.make_async_copy(rows[b2], agg_sh.at[sidx(0)],
                                      s_sems[b2]).wait()
                pltpu.async_copy(y_ref.at[gidx(g + KLOOK)], rows[b2],
                                 g_sems[b2])
        return carry

    lax.fori_loop(0, G // NBUF, group, 0)
    for b in range(NBUF):
        pltpu.make_async_copy(rows[b], agg_sh.at[sidx(0)], s_sems[b]).wait()
    if deg is not None:
        ones_v, deg_sh, d_sem, _ = deg

        def dwait(i, carry):
            pltpu.make_async_copy(ones_v, deg_sh.at[sidx(0)], d_sem).wait()
            return carry

        lax.fori_loop(0, G // 2, dwait, 0)


def _make_sc_layer1(n_nodes, half, slabs_per_tile):
    G = slabs_per_tile
    NPAD = _npad(n_nodes)
    ZR = NPAD // NS

    def body(*args):
        (ys_hbm, src_hbm, dst_hbm, zd_hbm, z16_hbm, ones_hbm,
         out_hbm, deg_hbm, srcbuf, dstbuf) = args[:10]
        rows = list(args[10:10 + NBUF])
        ones_v, agg_sh, deg_sh = args[10 + NBUF:13 + NBUF]
        g_sems = list(args[13 + NBUF:13 + 2 * NBUF])
        s_sems = list(args[13 + 2 * NBUF:13 + 3 * NBUF])
        dsem = args[13 + 3 * NBUF]
        c = lax.axis_index("c")
        s = lax.axis_index("s")

        pltpu.sync_copy(zd_hbm, agg_sh.at[pl.ds(s * ZR, ZR)])
        pltpu.sync_copy(z16_hbm, deg_sh.at[pl.ds(s * ZR, ZR)])
        pltpu.sync_copy(ones_hbm, ones_v)
        pltpu.sync_copy(src_hbm.at[pl.ds(s * G, G)], srcbuf)
        pltpu.sync_copy(dst_hbm.at[pl.ds(s * G, G)], dstbuf)
        plsc.subcore_barrier()

        hG = G // 2

        def cond_fn(g):
            return lax.select(c == 0, g < hG, g >= hG)

        _segsum_loop(ys_hbm.at[c], srcbuf, dstbuf, rows, g_sems, s_sems,
                     agg_sh, G, deg=(ones_v, deg_sh, dsem, cond_fn))

        plsc.subcore_barrier()
        pltpu.sync_copy(agg_sh.at[pl.ds(s * ZR, ZR)],
                        out_hbm.at[c, pl.ds(s * ZR, ZR)])
        pltpu.sync_copy(deg_sh.at[pl.ds(s * ZR, ZR)],
                        deg_hbm.at[c, pl.ds(s * ZR, ZR)])

    out_type = [
        jax.ShapeDtypeStruct((NC, NPAD, half), jnp.bfloat16),
        jax.ShapeDtypeStruct((NC, NPAD, DEGW), jnp.float32),
    ]
    scratch = (
        [pltpu.VMEM((G, SLAB), jnp.int32)] * 2
        + [pltpu.VMEM((SLAB, half), jnp.bfloat16)] * NBUF
        + [pltpu.VMEM((SLAB, DEGW), jnp.float32)]
        + [pltpu.VMEM_SHARED((NPAD, half), jnp.bfloat16)]
        + [pltpu.VMEM_SHARED((NPAD, DEGW), jnp.float32)]
        + [pltpu.SemaphoreType.DMA] * (2 * NBUF + 1)
    )
    mesh = plsc.VectorSubcoreMesh(core_axis_name="c", subcore_axis_name="s")
    return pl.kernel(
        body, out_type=out_type, mesh=mesh, scratch_types=scratch,
        compiler_params=pltpu.CompilerParams(use_tc_tiling_on_sc=False))


def _make_sc_layer2(n_nodes, d, slabs_per_tile):
    G = slabs_per_tile
    NPAD = _npad(n_nodes)
    ZR = NPAD // NS

    def body(*args):
        y_hbm, src_hbm, dst_hbm, zd_hbm, out_hbm, srcbuf, dstbuf = args[:7]
        rows = list(args[7:7 + NBUF])
        agg_sh = args[7 + NBUF]
        g_sems = list(args[8 + NBUF:8 + 2 * NBUF])
        s_sems = list(args[8 + 2 * NBUF:8 + 3 * NBUF])
        c = lax.axis_index("c")
        s = lax.axis_index("s")
        tid = c * NS + s

        pltpu.sync_copy(zd_hbm, agg_sh.at[pl.ds(s * ZR, ZR)])
        pltpu.sync_copy(src_hbm.at[pl.ds(tid * G, G)], srcbuf)
        pltpu.sync_copy(dst_hbm.at[pl.ds(tid * G, G)], dstbuf)
        plsc.subcore_barrier()

        _segsum_loop(y_hbm, srcbuf, dstbuf, rows, g_sems, s_sems, agg_sh, G)

        plsc.subcore_barrier()
        pltpu.sync_copy(agg_sh.at[pl.ds(s * ZR, ZR)],
                        out_hbm.at[c, pl.ds(s * ZR, ZR)])

    out_type = jax.ShapeDtypeStruct((NC, NPAD, d), jnp.bfloat16)
    scratch = (
        [pltpu.VMEM((G, SLAB), jnp.int32)] * 2
        + [pltpu.VMEM((SLAB, d), jnp.bfloat16)] * NBUF
        + [pltpu.VMEM_SHARED((NPAD, d), jnp.bfloat16)]
        + [pltpu.SemaphoreType.DMA] * (2 * NBUF)
    )
    mesh = plsc.VectorSubcoreMesh(core_axis_name="c", subcore_axis_name="s")
    return pl.kernel(
        body, out_type=out_type, mesh=mesh, scratch_types=scratch,
        compiler_params=pltpu.CompilerParams(use_tc_tiling_on_sc=False))


def kernel(x, edge_index, W1l, W1r, b1, W2l, W2r, b2):
    n, d_in = x.shape
    d_hid = W1l.shape[1]
    n_cls = W2l.shape[1]
    e = edge_index.shape[1]

    src = edge_index[0].astype(jnp.int32)
    dst = edge_index[1].astype(jnp.int32)

    unit = NC * NS * SLAB * NBUF
    e_pad = -(-e // unit) * unit
    pad = e_pad - e
    npad = _npad(n)
    pad_ar = jnp.arange(pad, dtype=jnp.int32)
    src_p = jnp.concatenate([src, pad_ar % n]).reshape(-1, SLAB)
    dst_p = jnp.concatenate([dst, n + pad_ar % (npad - n)]).reshape(-1, SLAB)
    g1 = e_pad // (NS * SLAB)
    g2 = e_pad // (NC * NS * SLAB)

    zr = npad // NS
    half = d_hid // 2
    z_half = jnp.zeros((zr, half), jnp.bfloat16)
    z_cls = jnp.zeros((zr, n_cls), jnp.bfloat16)
    z16 = jnp.zeros((zr, DEGW), jnp.float32)
    ones16 = jnp.ones((SLAB, DEGW), jnp.float32)

    y1s, self1 = pl.pallas_call(
        _proj_body,
        out_shape=[
            jax.ShapeDtypeStruct((NC, n, half), jnp.bfloat16),
            jax.ShapeDtypeStruct((n, d_hid), jnp.float32),
        ],
    )(x, W1l, W1r, b1.reshape(1, -1))

    sc1 = _make_sc_layer1(n, half, g1)
    s1p, degp = sc1(y1s, src_p, dst_p, z_half, z16, ones16)

    y2, self2 = pl.pallas_call(
        _mid_body,
        out_shape=[
            jax.ShapeDtypeStruct((n, n_cls), jnp.bfloat16),
            jax.ShapeDtypeStruct((n, n_cls), jnp.float32),
        ],
    )(s1p, degp, self1, W2l, W2r, b2.reshape(1, -1))

    sc2 = _make_sc_layer2(n, n_cls, g2)
    s2p = sc2(y2, src_p, dst_p, z_cls)

    out = pl.pallas_call(
        _final_body,
        out_shape=jax.ShapeDtypeStruct((n, n_cls), jnp.float32),
    )(s2p, degp, self2)
    return out


# speedup vs baseline: 1.2739x; 1.0394x over previous
import jax
import jax.numpy as jnp
from jax import lax
from jax.experimental import pallas as pl
from jax.experimental.pallas import tpu as pltpu
from jax.experimental.pallas import tpu_sc as plsc

NC = 2
NS = 16
SLAB = 128
NBUF = 4
KLOOK = 2
DEGW = 8


def _make_proj_body(n, npad, erows, rpad):
    def body(x_ref, ei_ref, wl_ref, wr_ref, b_ref, ys_ref, self_ref,
             src_ref, dst_ref):
        xx = x_ref[...]
        y = jnp.dot(xx, wl_ref[...], preferred_element_type=jnp.float32)
        half = y.shape[1] // 2
        yb = y.astype(jnp.bfloat16)
        ys_ref[0] = yb[:, :half]
        ys_ref[1] = yb[:, half:]
        self_ref[...] = (
            jnp.dot(xx, wr_ref[...], preferred_element_type=jnp.float32)
            + b_ref[...]
        )
        src_ref[0:erows] = ei_ref[0]
        dst_ref[0:erows] = ei_ref[1]
        pr = rpad - erows
        if pr:
            flat = (lax.broadcasted_iota(jnp.int32, (pr, SLAB), 0) * SLAB
                    + lax.broadcasted_iota(jnp.int32, (pr, SLAB), 1))
            src_ref[erows:rpad] = flat % n
            dst_ref[erows:rpad] = n + flat % (npad - n)

    return body


def _mid_body(s1_ref, deg_ref, self1_ref, wl_ref, wr_ref, b_ref, y2_ref,
              self2_ref):
    nn = self1_ref.shape[0]
    d = deg_ref[0, :nn, 0:1] + deg_ref[1, :nn, 0:1]
    rec = 1.0 / jnp.maximum(d, 1.0)
    s1 = jnp.concatenate([s1_ref[0, :nn], s1_ref[1, :nn]],
                         axis=1).astype(jnp.float32)
    h = jnp.maximum(s1 * rec + self1_ref[...], 0.0)
    y2_ref[...] = jnp.dot(h, wl_ref[...],
                          preferred_element_type=jnp.float32).astype(jnp.bfloat16)
    self2_ref[...] = (
        jnp.dot(h, wr_ref[...], preferred_element_type=jnp.float32) + b_ref[...]
    )


def _final_body(s2_ref, deg_ref, self2_ref, out_ref):
    nn = self2_ref.shape[0]
    d = deg_ref[0, :nn, 0:1] + deg_ref[1, :nn, 0:1]
    rec = 1.0 / jnp.maximum(d, 1.0)
    s2 = s2_ref[0, :nn].astype(jnp.float32) + s2_ref[1, :nn].astype(jnp.float32)
    z = s2 * rec + self2_ref[...]
    m = jnp.max(z, axis=1, keepdims=True)
    zs = z - m
    lse = jnp.log(jnp.sum(jnp.exp(zs), axis=1, keepdims=True))
    out_ref[...] = zs - lse


def _npad(n_nodes):
    return ((n_nodes + 1 + 127) // 128) * 128


def _segsum_loop(y_ref, srcbuf, dstbuf, rows, g_sems, s_sems,
                 agg_sh, n_slabs, deg=None):
    G = n_slabs

    def gidx(g):
        return srcbuf.at[g]

    def sidx(g):
        return dstbuf.at[g]

    for b in range(NBUF):
        pltpu.async_copy(y_ref.at[gidx(b)], rows[b], g_sems[b])

    def group(i, carry):
        for b in range(NBUF):
            g = i * NBUF + b
            pltpu.make_async_copy(y_ref.at[gidx(g)], rows[b],
                                  g_sems[b]).wait()
            pltpu.async_copy(rows[b], agg_sh.at[sidx(g)], s_sems[b],
                             add=True)
            if deg is not None:
                ones_v, deg_sh, d_sem, cond_fn = deg

                @pl.when(cond_fn(g))
                def _():
                    pltpu.async_copy(ones_v, deg_sh.at[sidx(g)], d_sem,
                                     add=True)

            b2 = (b + KLOOK) % NBUF

            @pl.when((g >= NBUF - KLOOK) & (g + KLOOK < G))
            def _():
                pltpu.make_async_copy(rows[b2], agg_sh.at[sidx(0)],
                                      s_sems[b2]).wait()
                pltpu.async_copy(y_ref.at[gidx(g + KLOOK)], rows[b2],
                                 g_sems[b2])
        return carry

    lax.fori_loop(0, G // NBUF, group, 0)
    for b in range(NBUF):
        pltpu.make_async_copy(rows[b], agg_sh.at[sidx(0)], s_sems[b]).wait()
    if deg is not None:
        ones_v, deg_sh, d_sem, _ = deg

        def dwait(i, carry):
            pltpu.make_async_copy(ones_v, deg_sh.at[sidx(0)], d_sem).wait()
            return carry

        lax.fori_loop(0, G // 2, dwait, 0)


def _make_sc_layer1(n_nodes, half, slabs_per_tile):
    G = slabs_per_tile
    NPAD = _npad(n_nodes)
    ZR = NPAD // NS

    def body(*args):
        (ys_hbm, src_hbm, dst_hbm, zd_hbm, z16_hbm, ones_hbm,
         out_hbm, deg_hbm, srcbuf, dstbuf) = args[:10]
        rows = list(args[10:10 + NBUF])
        ones_v, agg_sh, deg_sh = args[10 + NBUF:13 + NBUF]
        g_sems = list(args[13 + NBUF:13 + 2 * NBUF])
        s_sems = list(args[13 + 2 * NBUF:13 + 3 * NBUF])
        dsem = args[13 + 3 * NBUF]
        c = lax.axis_index("c")
        s = lax.axis_index("s")

        pltpu.sync_copy(zd_hbm, agg_sh.at[pl.ds(s * ZR, ZR)])
        pltpu.sync_copy(z16_hbm, deg_sh.at[pl.ds(s * ZR, ZR)])
        pltpu.sync_copy(ones_hbm, ones_v)
        pltpu.sync_copy(src_hbm.at[pl.ds(s * G, G)], srcbuf)
        pltpu.sync_copy(dst_hbm.at[pl.ds(s * G, G)], dstbuf)
        plsc.subcore_barrier()

        hG = G // 2

        def cond_fn(g):
            return lax.select(c == 0, g < hG, g >= hG)

        _segsum_loop(ys_hbm.at[c], srcbuf, dstbuf, rows, g_sems, s_sems,
                     agg_sh, G, deg=(ones_v, deg_sh, dsem, cond_fn))

        plsc.subcore_barrier()
        pltpu.sync_copy(agg_sh.at[pl.ds(s * ZR, ZR)],
                        out_hbm.at[c, pl.ds(s * ZR, ZR)])
        pltpu.sync_copy(deg_sh.at[pl.ds(s * ZR, ZR)],
                        deg_hbm.at[c, pl.ds(s * ZR, ZR)])

    out_type = [
        jax.ShapeDtypeStruct((NC, NPAD, half), jnp.bfloat16),
        jax.ShapeDtypeStruct((NC, NPAD, DEGW), jnp.float32),
    ]
    scratch = (
        [pltpu.VMEM((G, SLAB), jnp.int32)] * 2
        + [pltpu.VMEM((SLAB, half), jnp.bfloat16)] * NBUF
        + [pltpu.VMEM((SLAB, DEGW), jnp.float32)]
        + [pltpu.VMEM_SHARED((NPAD, half), jnp.bfloat16)]
        + [pltpu.VMEM_SHARED((NPAD, DEGW), jnp.float32)]
        + [pltpu.SemaphoreType.DMA] * (2 * NBUF + 1)
    )
    mesh = plsc.VectorSubcoreMesh(core_axis_name="c", subcore_axis_name="s")
    return pl.kernel(
        body, out_type=out_type, mesh=mesh, scratch_types=scratch,
        compiler_params=pltpu.CompilerParams(use_tc_tiling_on_sc=False))


def _make_sc_layer2(n_nodes, d, slabs_per_tile):
    G = slabs_per_tile
    NPAD = _npad(n_nodes)
    ZR = NPAD // NS

    def body(*args):
        y_hbm, src_hbm, dst_hbm, zd_hbm, out_hbm, srcbuf, dstbuf = args[:7]
        rows = list(args[7:7 + NBUF])
        agg_sh = args[7 + NBUF]
        g_sems = list(args[8 + NBUF:8 + 2 * NBUF])
        s_sems = list(args[8 + 2 * NBUF:8 + 3 * NBUF])
        c = lax.axis_index("c")
        s = lax.axis_index("s")
        tid = c * NS + s

        pltpu.sync_copy(zd_hbm, agg_sh.at[pl.ds(s * ZR, ZR)])
        pltpu.sync_copy(src_hbm.at[pl.ds(tid * G, G)], srcbuf)
        pltpu.sync_copy(dst_hbm.at[pl.ds(tid * G, G)], dstbuf)
        plsc.subcore_barrier()

        _segsum_loop(y_hbm, srcbuf, dstbuf, rows, g_sems, s_sems, agg_sh, G)

        plsc.subcore_barrier()
        pltpu.sync_copy(agg_sh.at[pl.ds(s * ZR, ZR)],
                        out_hbm.at[c, pl.ds(s * ZR, ZR)])

    out_type = jax.ShapeDtypeStruct((NC, NPAD, d), jnp.bfloat16)
    scratch = (
        [pltpu.VMEM((G, SLAB), jnp.int32)] * 2
        + [pltpu.VMEM((SLAB, d), jnp.bfloat16)] * NBUF
        + [pltpu.VMEM_SHARED((NPAD, d), jnp.bfloat16)]
        + [pltpu.SemaphoreType.DMA] * (2 * NBUF)
    )
    mesh = plsc.VectorSubcoreMesh(core_axis_name="c", subcore_axis_name="s")
    return pl.kernel(
        body, out_type=out_type, mesh=mesh, scratch_types=scratch,
        compiler_params=pltpu.CompilerParams(use_tc_tiling_on_sc=False))


def kernel(x, edge_index, W1l, W1r, b1, W2l, W2r, b2):
    n, d_in = x.shape
    d_hid = W1l.shape[1]
    n_cls = W2l.shape[1]
    e = edge_index.shape[1]

    unit = NC * NS * SLAB * NBUF
    e_pad = -(-e // unit) * unit
    npad = _npad(n)
    erows = e // SLAB
    rpad = e_pad // SLAB
    ei3 = edge_index.astype(jnp.int32).reshape(2, erows, SLAB)
    g1 = e_pad // (NS * SLAB)
    g2 = e_pad // (NC * NS * SLAB)

    zr = npad // NS
    half = d_hid // 2
    z_half = jnp.zeros((zr, half), jnp.bfloat16)
    z_cls = jnp.zeros((zr, n_cls), jnp.bfloat16)
    z16 = jnp.zeros((zr, DEGW), jnp.float32)
    ones16 = jnp.ones((SLAB, DEGW), jnp.float32)

    y1s, self1, src_p, dst_p = pl.pallas_call(
        _make_proj_body(n, npad, erows, rpad),
        out_shape=[
            jax.ShapeDtypeStruct((NC, n, half), jnp.bfloat16),
            jax.ShapeDtypeStruct((n, d_hid), jnp.float32),
            jax.ShapeDtypeStruct((rpad, SLAB), jnp.int32),
            jax.ShapeDtypeStruct((rpad, SLAB), jnp.int32),
        ],
    )(x, ei3, W1l, W1r, b1.reshape(1, -1))

    sc1 = _make_sc_layer1(n, half, g1)
    s1p, degp = sc1(y1s, src_p, dst_p, z_half, z16, ones16)

    y2, self2 = pl.pallas_call(
        _mid_body,
        out_shape=[
            jax.ShapeDtypeStruct((n, n_cls), jnp.bfloat16),
            jax.ShapeDtypeStruct((n, n_cls), jnp.float32),
        ],
    )(s1p, degp, self1, W2l, W2r, b2.reshape(1, -1))

    sc2 = _make_sc_layer2(n, n_cls, g2)
    s2p = sc2(y2, src_p, dst_p, z_cls)

    out = pl.pallas_call(
        _final_body,
        out_shape=jax.ShapeDtypeStruct((n, n_cls), jnp.float32),
    )(s2p, degp, self2)
    return out
